# baseline (speedup 1.0000x reference)
import jax
import jax.numpy as jnp
from jax import lax
from jax.experimental import pallas as pl
from jax.experimental.pallas import tpu as pltpu
from jax.experimental.pallas import tpu_sc as plsc

V = 100000
NA = 64
ROW_F = 2 * NA
B = 16384
L = 16
NW = 32
PER_W = B // NW
BLK = 128
NBLK = B // BLK
CHUNK = BLK
N_CHUNKS = PER_W // CHUNK
CB = 512


def _prep_body(crT_ref, ciT_ref, cb_ref):
  ar = crT_ref[...]
  ai = ciT_ref[...]
  ss = jnp.sum(ar * ar + ai * ai, axis=0)
  sc = lax.rsqrt(ss)
  arn = ar * sc[None, :]
  ain = ai * sc[None, :]
  cb_ref[...] = jnp.concatenate([arn.T, ain.T], axis=1)


@jax.jit
def _prep(crT, ciT):
  grid = (-(-V // CB),)
  return pl.pallas_call(
      _prep_body,
      grid=grid,
      in_specs=[
          pl.BlockSpec((NA, CB), lambda i: (0, i)),
          pl.BlockSpec((NA, CB), lambda i: (0, i)),
      ],
      out_specs=pl.BlockSpec((CB, ROW_F), lambda i: (i, 0)),
      out_shape=jax.ShapeDtypeStruct((V, ROW_F), jnp.float32),
  )(crT, ciT)


def _sc_body(idx_hbm, cb_hbm, out_hbm,
             idx_v, rows0, rows1, outb0, outb1,
             sem_g0, sem_g1, sem_o0, sem_o1):
  nc = 2
  wid = lax.axis_index("s") * nc + lax.axis_index("c")
  base = wid * PER_W
  iota = lax.iota(jnp.int32, L)

  rows = (rows0, rows1)
  outb = (outb0, outb1)
  sem_g = (sem_g0, sem_g1)
  sem_o = (sem_o0, sem_o1)

  pltpu.sync_copy(idx_hbm.at[pl.ds(base, PER_W)], idx_v)

  def fire_gather(c):
    p = c % 2
    pltpu.make_async_copy(
        cb_hbm.at[idx_v.at[pl.ds(c * CHUNK, CHUNK)]], rows[p],
        sem_g[p]).start()

  def drain_gather(c):
    p = c % 2
    pltpu.make_async_copy(cb_hbm.at[pl.ds(0, CHUNK), :], rows[p],
                          sem_g[p]).wait()

  def compute(c):
    p = c % 2

    def row_body(j, carry):
      for k in range(NA // L):
        vr = rows[p][j, pl.ds(L * k, L)]
        vi = rows[p][j, pl.ds(NA + L * k, L)]
        col = j + (iota + (L * k)) * (2 * BLK)
        plsc.store_scatter(outb[p], [col], vr)
        plsc.store_scatter(outb[p], [col + BLK], vi)
      return carry

    lax.fori_loop(0, CHUNK, row_body, 0, unroll=4)

  def fire_out(c):
    p = c % 2
    tglob = wid * N_CHUNKS + c
    for a in range(NA):
      pltpu.make_async_copy(
          outb[p].at[pl.ds(a * (2 * BLK), 2 * BLK)],
          out_hbm.at[pl.ds(a * (2 * BLK * NBLK) + tglob * (2 * BLK),
                           2 * BLK)],
          sem_o[p]).start()

  def drain_out(c):
    p = c % 2
    pltpu.make_async_copy(out_hbm.at[pl.ds(0, ROW_F * BLK)], outb[p],
                          sem_o[p]).wait()

  fire_gather(0)
  for c in range(N_CHUNKS):
    if c + 1 < N_CHUNKS:
      fire_gather(c + 1)
    drain_gather(c)
    if c >= 2:
      drain_out(c - 2)
    compute(c)
    fire_out(c)
  drain_out(N_CHUNKS - 2)
  drain_out(N_CHUNKS - 1)


@jax.jit
def _sc_call(beam_index, cb):
  mesh = plsc.VectorSubcoreMesh(core_axis_name="c", subcore_axis_name="s")
  f = pl.kernel(
      _sc_body,
      out_type=jax.ShapeDtypeStruct((B * ROW_F,), jnp.float32),
      mesh=mesh,
      compiler_params=pltpu.CompilerParams(
          needs_layout_passes=False, use_tc_tiling_on_sc=True),
      scratch_types=[
          pltpu.VMEM((PER_W,), jnp.int32),
          pltpu.VMEM((CHUNK, ROW_F), jnp.float32),
          pltpu.VMEM((CHUNK, ROW_F), jnp.float32),
          pltpu.VMEM((ROW_F * BLK,), jnp.float32),
          pltpu.VMEM((ROW_F * BLK,), jnp.float32),
          pltpu.SemaphoreType.DMA,
          pltpu.SemaphoreType.DMA,
          pltpu.SemaphoreType.DMA,
          pltpu.SemaphoreType.DMA,
      ],
  )
  return f(beam_index, cb)


def kernel(beam_index, codebook_real, codebook_imag):
  cb = _prep(codebook_real.T, codebook_imag.T)
  out = _sc_call(beam_index, cb)
  out = out.reshape(NA, NBLK, 2, BLK)
  out = out.transpose(1, 3, 0, 2)
  return out.reshape(B, NA, 2)


# speedup vs baseline: 1.7739x; 1.7739x over previous
import jax
import jax.numpy as jnp
from jax import lax
from jax.experimental import pallas as pl
from jax.experimental.pallas import tpu as pltpu
from jax.experimental.pallas import tpu_sc as plsc

V = 100000
NA = 64
ROW_F = 2 * NA
B = 16384
L = 16
NW = 32
PER_W = B // NW
BLK = 128
NBLK = B // BLK
CHUNK = BLK
N_CHUNKS = PER_W // CHUNK
CB = 4096


def _prep_body(crT_ref, ciT_ref, cb_ref):
  ar = crT_ref[...]
  ai = ciT_ref[...]
  ss = jnp.sum(ar * ar + ai * ai, axis=0)
  sc = lax.rsqrt(ss)
  arn = ar * sc[None, :]
  ain = ai * sc[None, :]
  cb_ref[...] = jnp.concatenate([arn.T, ain.T], axis=1)


@jax.jit
def _prep(crT, ciT):
  grid = (-(-V // CB),)
  return pl.pallas_call(
      _prep_body,
      grid=grid,
      in_specs=[
          pl.BlockSpec((NA, CB), lambda i: (0, i)),
          pl.BlockSpec((NA, CB), lambda i: (0, i)),
      ],
      out_specs=pl.BlockSpec((CB, ROW_F), lambda i: (i, 0)),
      out_shape=jax.ShapeDtypeStruct((V, ROW_F), jnp.float32),
  )(crT, ciT)


def _sc_body(idx_hbm, cb_hbm, out_hbm,
             idx_v, rows0, rows1, outb0, outb1,
             sem_g0, sem_g1, sem_o0, sem_o1):
  nc = 2
  wid = lax.axis_index("s") * nc + lax.axis_index("c")
  base = wid * PER_W
  iota = lax.iota(jnp.int32, L)

  rows = (rows0, rows1)
  outb = (outb0, outb1)
  sem_g = (sem_g0, sem_g1)
  sem_o = (sem_o0, sem_o1)

  pltpu.sync_copy(idx_hbm.at[pl.ds(base, PER_W)], idx_v)

  def fire_gather(c):
    p = c % 2
    pltpu.make_async_copy(
        cb_hbm.at[idx_v.at[pl.ds(c * CHUNK, CHUNK)]], rows[p],
        sem_g[p]).start()

  def drain_gather(c):
    p = c % 2
    pltpu.make_async_copy(cb_hbm.at[pl.ds(0, CHUNK), :], rows[p],
                          sem_g[p]).wait()

  def compute(c):
    p = c % 2

    def row_body(j, carry):
      for k in range(NA // L):
        vr = rows[p][j, pl.ds(L * k, L)]
        vi = rows[p][j, pl.ds(NA + L * k, L)]
        col = j + (iota + (L * k)) * (2 * BLK)
        plsc.store_scatter(outb[p], [col], vr)
        plsc.store_scatter(outb[p], [col + BLK], vi)
      return carry

    lax.fori_loop(0, CHUNK, row_body, 0, unroll=4)

  def fire_out(c):
    p = c % 2
    tglob = wid * N_CHUNKS + c
    for a in range(NA):
      pltpu.make_async_copy(
          outb[p].at[pl.ds(a * (2 * BLK), 2 * BLK)],
          out_hbm.at[pl.ds(a * (2 * BLK * NBLK) + tglob * (2 * BLK),
                           2 * BLK)],
          sem_o[p]).start()

  def drain_out(c):
    p = c % 2
    pltpu.make_async_copy(out_hbm.at[pl.ds(0, ROW_F * BLK)], outb[p],
                          sem_o[p]).wait()

  fire_gather(0)
  for c in range(N_CHUNKS):
    if c + 1 < N_CHUNKS:
      fire_gather(c + 1)
    drain_gather(c)
    if c >= 2:
      drain_out(c - 2)
    compute(c)
    fire_out(c)
  drain_out(N_CHUNKS - 2)
  drain_out(N_CHUNKS - 1)


@jax.jit
def _sc_call(beam_index, cb):
  mesh = plsc.VectorSubcoreMesh(core_axis_name="c", subcore_axis_name="s")
  f = pl.kernel(
      _sc_body,
      out_type=jax.ShapeDtypeStruct((B * ROW_F,), jnp.float32),
      mesh=mesh,
      compiler_params=pltpu.CompilerParams(
          needs_layout_passes=False, use_tc_tiling_on_sc=True),
      scratch_types=[
          pltpu.VMEM((PER_W,), jnp.int32),
          pltpu.VMEM((CHUNK, ROW_F), jnp.float32),
          pltpu.VMEM((CHUNK, ROW_F), jnp.float32),
          pltpu.VMEM((ROW_F * BLK,), jnp.float32),
          pltpu.VMEM((ROW_F * BLK,), jnp.float32),
          pltpu.SemaphoreType.DMA,
          pltpu.SemaphoreType.DMA,
          pltpu.SemaphoreType.DMA,
          pltpu.SemaphoreType.DMA,
      ],
  )
  return f(beam_index, cb)


def kernel(beam_index, codebook_real, codebook_imag):
  cb = _prep(codebook_real.T, codebook_imag.T)
  out = _sc_call(beam_index, cb)
  out = out.reshape(NA, NBLK, 2, BLK)
  out = out.transpose(1, 3, 0, 2)
  return out.reshape(B, NA, 2)
